# baseline (device time: 35026 ns/iter reference)
import os

import jax
import jax.numpy as jnp
from jax import lax
from jax.experimental import pallas as pl
from jax.experimental.pallas import tpu as pltpu

_MODE = os.environ.get("GENDIST_MODE", "full")

N_DEV = 8
B_LOC = 2
SQ = 256
SKV = 256
H_GLOBAL = 32
H_LOC = 4
DH = 64
D_MODEL = 512
BLK = 64
HD = H_LOC * DH
NB = SQ // BLK


def kernel(x, Wq, K_ext, V_ext, Wo):
    my = lax.axis_index("i")
    k_loc = lax.dynamic_slice(
        jnp.reshape(K_ext, (N_DEV * B_LOC, SKV, H_GLOBAL * DH)),
        (B_LOC * my, 0, 0), (B_LOC, SKV, H_GLOBAL * DH),
    )
    v_loc = lax.dynamic_slice(
        jnp.reshape(V_ext, (N_DEV * B_LOC, SKV, H_GLOBAL * DH)),
        (B_LOC * my, 0, 0), (B_LOC, SKV, H_GLOBAL * DH),
    )

    def body(x_ref, wq_ref, wo_ref, k_bf, v_bf, out_ref,
             stage_wq, stage_wo, stage_sc, comm_wq, comm_wo, comm_sc,
             send_wq, recv_wq, send_wo, recv_wo, send_sc, recv_sc):
        my_pos = lax.axis_index("i")

        wq = wq_ref[:] * 0.125
        wo = wo_ref[:]
        sq = jnp.max(jnp.abs(wq), axis=0, keepdims=True) / 127.0
        so = jnp.max(jnp.abs(wo), axis=0, keepdims=True) / 127.0
        stage_wq[:] = jnp.clip(jnp.round(wq / sq), -127.0, 127.0
                               ).astype(jnp.int8)
        stage_wo[:] = jnp.clip(jnp.round(wo / so), -127.0, 127.0
                               ).astype(jnp.int8)
        stage_sc[0:1, 0:HD] = sq.astype(jnp.bfloat16)
        stage_sc[1:2, :] = so.astype(jnp.bfloat16)

        x_all = jnp.reshape(x_ref[:], (B_LOC * SQ, D_MODEL)
                            ).astype(jnp.bfloat16)

        def compute(first, origin, wq_i8, wo_i8, sq_row, so_row):
            wq16 = wq_i8.astype(jnp.bfloat16) * sq_row
            wo16 = wo_i8.astype(jnp.bfloat16) * so_row
            q_all = jnp.dot(x_all, wq16,
                            preferred_element_type=jnp.float32
                            ).astype(jnp.bfloat16)
            k_blks = []
            v_blks = []
            strips = []
            for b in range(B_LOC):
                q_b = q_all[b * SQ:(b + 1) * SQ]
                k_blks.append(
                    k_bf[b, :, pl.ds(origin * HD, HD)].astype(jnp.bfloat16))
                v_blks.append(
                    v_bf[b, :, pl.ds(origin * HD, HD)].astype(jnp.bfloat16))
                for hh in range(H_LOC):
                    q_bh = q_b[:, hh * DH:(hh + 1) * DH]
                    k_bh = k_blks[b][:, hh * DH:(hh + 1) * DH]
                    for z in range(NB):
                        strips.append(lax.dot_general(
                            q_bh[z * BLK:(z + 1) * BLK],
                            k_bh[z * BLK:(z + 1) * BLK],
                            (((1,), (1,)), ((), ())),
                            preferred_element_type=jnp.float32,
                        ))
            s = jnp.concatenate(strips, axis=0)
            w = jnp.exp(s)
            p = (w / jnp.sum(w, axis=1, keepdims=True)).astype(jnp.bfloat16)
            ctx_bs = []
            for b in range(B_LOC):
                ctx_heads = []
                for hh in range(H_LOC):
                    v_bh = v_blks[b][:, hh * DH:(hh + 1) * DH]
                    ctx_blocks = []
                    for z in range(NB):
                        i = (b * H_LOC + hh) * NB + z
                        ctx_blocks.append(jnp.dot(
                            p[i * BLK:(i + 1) * BLK],
                            v_bh[z * BLK:(z + 1) * BLK],
                            preferred_element_type=jnp.float32,
                        ).astype(jnp.bfloat16))
                    ctx_heads.append(jnp.concatenate(ctx_blocks, axis=0))
                ctx_bs.append(jnp.concatenate(ctx_heads, axis=1))
            ctx_all = jnp.concatenate(ctx_bs, axis=0)
            contrib = jnp.dot(ctx_all, wo16,
                              preferred_element_type=jnp.float32)
            contrib = jnp.reshape(contrib, (B_LOC, SQ, D_MODEL))
            if first:
                out_ref[:] = contrib
            else:
                out_ref[:] = out_ref[:] + contrib

        def own_block():
            compute(True, my_pos, stage_wq[:], stage_wo[:],
                    stage_sc[0:1, 0:HD], stage_sc[1:2, :])

        if _MODE == "compute":
            own_block()
            for s in range(N_DEV - 1):
                origin = lax.rem(my_pos + 1 + s, N_DEV)
                compute(False, origin, stage_wq[:], stage_wo[:],
                        stage_sc[0:1, 0:HD], stage_sc[1:2, :])
            return

        barrier = pltpu.get_barrier_semaphore()
        for k in range(1, N_DEV):
            pl.semaphore_signal(
                barrier, inc=1,
                device_id=(lax.rem(my_pos + k, N_DEV),),
                device_id_type=pl.DeviceIdType.MESH,
            )
        pl.semaphore_wait(barrier, N_DEV - 1)

        sends = []
        for s in range(N_DEV - 1):
            t = lax.rem(my_pos + 1 + s, N_DEV)
            slot = N_DEV - 2 - s
            for stage, comm, ssem, rsem in (
                (stage_sc, comm_sc, send_sc, recv_sc),
                (stage_wq, comm_wq, send_wq, recv_wq),
                (stage_wo, comm_wo, send_wo, recv_wo),
            ):
                r = pltpu.make_async_remote_copy(
                    src_ref=stage,
                    dst_ref=comm.at[slot],
                    send_sem=ssem.at[s],
                    recv_sem=rsem.at[slot],
                    device_id=(t,),
                    device_id_type=pl.DeviceIdType.MESH,
                )
                r.start()
                sends.append(r)

        own_block()

        for s in reversed(range(N_DEV - 1)):
            for comm, ssem, rsem in (
                (comm_sc, send_sc, recv_sc),
                (comm_wq, send_wq, recv_wq),
                (comm_wo, send_wo, recv_wo),
            ):
                recv = pltpu.make_async_remote_copy(
                    src_ref=comm.at[s],
                    dst_ref=comm.at[s],
                    send_sem=ssem.at[s],
                    recv_sem=rsem.at[s],
                    device_id=(my_pos,),
                    device_id_type=pl.DeviceIdType.MESH,
                )
                recv.wait_recv()
            if _MODE != "comm":
                origin = lax.rem(my_pos + 1 + s, N_DEV)
                compute(False, origin, comm_wq[s], comm_wo[s],
                        comm_sc[s, 0:1, 0:HD], comm_sc[s, 1:2, :])

        for r in sends:
            r.wait_send()

    return pl.pallas_call(
        body,
        out_shape=jax.ShapeDtypeStruct((B_LOC, SQ, D_MODEL), jnp.float32),
        in_specs=[
            pl.BlockSpec(memory_space=pltpu.VMEM),
            pl.BlockSpec(memory_space=pltpu.VMEM),
            pl.BlockSpec(memory_space=pltpu.VMEM),
            pl.BlockSpec(memory_space=pltpu.VMEM),
            pl.BlockSpec(memory_space=pltpu.VMEM),
        ],
        out_specs=pl.BlockSpec(memory_space=pltpu.VMEM),
        scratch_shapes=[
            pltpu.VMEM((D_MODEL, HD), jnp.int8),
            pltpu.VMEM((HD, D_MODEL), jnp.int8),
            pltpu.VMEM((8, D_MODEL), jnp.bfloat16),
            pltpu.VMEM((N_DEV - 1, D_MODEL, HD), jnp.int8),
            pltpu.VMEM((N_DEV - 1, HD, D_MODEL), jnp.int8),
            pltpu.VMEM((N_DEV - 1, 8, D_MODEL), jnp.bfloat16),
            pltpu.SemaphoreType.DMA((N_DEV - 1,)),
            pltpu.SemaphoreType.DMA((N_DEV - 1,)),
            pltpu.SemaphoreType.DMA((N_DEV - 1,)),
            pltpu.SemaphoreType.DMA((N_DEV - 1,)),
            pltpu.SemaphoreType.DMA((N_DEV - 1,)),
            pltpu.SemaphoreType.DMA((N_DEV - 1,)),
        ],
        compiler_params=pltpu.CompilerParams(
            collective_id=None if _MODE == "compute" else 0
        ),
    )(x, Wq, Wo, k_loc, v_loc)


# device time: 32580 ns/iter; 1.0751x vs baseline; 1.0751x over previous
import os

import jax
import jax.numpy as jnp
from jax import lax
from jax.experimental import pallas as pl
from jax.experimental.pallas import tpu as pltpu

_MODE = os.environ.get("GENDIST_MODE", "full")

N_DEV = 8
B_LOC = 2
SQ = 256
SKV = 256
H_GLOBAL = 32
H_LOC = 4
DH = 64
D_MODEL = 512
BLK = 64
HD = H_LOC * DH
NB = SQ // BLK


def kernel(x, Wq, K_ext, V_ext, Wo):
    my = lax.axis_index("i")
    k_loc = lax.dynamic_slice(
        jnp.reshape(K_ext, (N_DEV * B_LOC, SKV, H_GLOBAL * DH)),
        (B_LOC * my, 0, 0), (B_LOC, SKV, H_GLOBAL * DH),
    ).astype(jnp.bfloat16)
    v_loc = lax.dynamic_slice(
        jnp.reshape(V_ext, (N_DEV * B_LOC, SKV, H_GLOBAL * DH)),
        (B_LOC * my, 0, 0), (B_LOC, SKV, H_GLOBAL * DH),
    ).astype(jnp.bfloat16)

    def body(x_ref, wq_ref, wo_ref, k_bf, v_bf, out_ref,
             stage_wq, stage_wo, stage_sc, comm_wq, comm_wo, comm_sc,
             send_wq, recv_wq, send_wo, recv_wo, send_sc, recv_sc):
        my_pos = lax.axis_index("i")

        if _MODE != "compute":
            barrier = pltpu.get_barrier_semaphore()
            for k in range(1, N_DEV):
                pl.semaphore_signal(
                    barrier, inc=1,
                    device_id=(lax.rem(my_pos + k, N_DEV),),
                    device_id_type=pl.DeviceIdType.MESH,
                )

        wq = wq_ref[:] * 0.125
        wo = wo_ref[:]
        sq = jnp.max(jnp.abs(wq), axis=0, keepdims=True) / 127.0
        so = jnp.max(jnp.abs(wo), axis=0, keepdims=True) / 127.0
        stage_wq[:] = jnp.clip(jnp.round(wq / sq), -127.0, 127.0
                               ).astype(jnp.int8)
        stage_wo[:] = jnp.clip(jnp.round(wo / so), -127.0, 127.0
                               ).astype(jnp.int8)
        stage_sc[0:1, 0:HD] = sq.astype(jnp.bfloat16)
        stage_sc[1:2, :] = so.astype(jnp.bfloat16)

        x_all = jnp.reshape(x_ref[:], (B_LOC * SQ, D_MODEL)
                            ).astype(jnp.bfloat16)

        def compute(first, origin, wq_i8, wo_i8, sq_row, so_row):
            wq16 = wq_i8.astype(jnp.bfloat16) * sq_row
            wo16 = wo_i8.astype(jnp.bfloat16) * so_row
            q_all = jnp.dot(x_all, wq16,
                            preferred_element_type=jnp.float32
                            ).astype(jnp.bfloat16)
            k_blks = []
            v_blks = []
            strips = []
            for b in range(B_LOC):
                q_b = q_all[b * SQ:(b + 1) * SQ]
                k_blks.append(k_bf[b, :, pl.ds(origin * HD, HD)])
                v_blks.append(v_bf[b, :, pl.ds(origin * HD, HD)])
                for hh in range(H_LOC):
                    q_bh = q_b[:, hh * DH:(hh + 1) * DH]
                    k_bh = k_blks[b][:, hh * DH:(hh + 1) * DH]
                    for z in range(NB):
                        strips.append(lax.dot_general(
                            q_bh[z * BLK:(z + 1) * BLK],
                            k_bh[z * BLK:(z + 1) * BLK],
                            (((1,), (1,)), ((), ())),
                            preferred_element_type=jnp.float32,
                        ))
            s = jnp.concatenate(strips, axis=0)
            w = jnp.exp(s)
            p = (w / jnp.sum(w, axis=1, keepdims=True)).astype(jnp.bfloat16)
            ctx_bs = []
            for b in range(B_LOC):
                ctx_heads = []
                for hh in range(H_LOC):
                    v_bh = v_blks[b][:, hh * DH:(hh + 1) * DH]
                    ctx_blocks = []
                    for z in range(NB):
                        i = (b * H_LOC + hh) * NB + z
                        ctx_blocks.append(jnp.dot(
                            p[i * BLK:(i + 1) * BLK],
                            v_bh[z * BLK:(z + 1) * BLK],
                            preferred_element_type=jnp.float32,
                        ).astype(jnp.bfloat16))
                    ctx_heads.append(jnp.concatenate(ctx_blocks, axis=0))
                ctx_bs.append(jnp.concatenate(ctx_heads, axis=1))
            ctx_all = jnp.concatenate(ctx_bs, axis=0)
            contrib = jnp.dot(ctx_all, wo16,
                              preferred_element_type=jnp.float32)
            contrib = jnp.reshape(contrib, (B_LOC, SQ, D_MODEL))
            if first:
                out_ref[:] = contrib
            else:
                out_ref[:] = out_ref[:] + contrib

        def own_block():
            compute(True, my_pos, stage_wq[:], stage_wo[:],
                    stage_sc[0:1, 0:HD], stage_sc[1:2, :])

        if _MODE == "compute":
            own_block()
            for s in range(N_DEV - 1):
                origin = lax.rem(my_pos + 1 + s, N_DEV)
                compute(False, origin, stage_wq[:], stage_wo[:],
                        stage_sc[0:1, 0:HD], stage_sc[1:2, :])
            return

        pl.semaphore_wait(pltpu.get_barrier_semaphore(), N_DEV - 1)

        sends = []
        for s in range(N_DEV - 1):
            t = lax.rem(my_pos + 1 + s, N_DEV)
            slot = N_DEV - 2 - s
            for stage, comm, ssem, rsem in (
                (stage_sc, comm_sc, send_sc, recv_sc),
                (stage_wq, comm_wq, send_wq, recv_wq),
                (stage_wo, comm_wo, send_wo, recv_wo),
            ):
                r = pltpu.make_async_remote_copy(
                    src_ref=stage,
                    dst_ref=comm.at[slot],
                    send_sem=ssem.at[s],
                    recv_sem=rsem.at[slot],
                    device_id=(t,),
                    device_id_type=pl.DeviceIdType.MESH,
                )
                r.start()
                sends.append(r)

        own_block()

        for s in reversed(range(N_DEV - 1)):
            for comm, ssem, rsem in (
                (comm_sc, send_sc, recv_sc),
                (comm_wq, send_wq, recv_wq),
                (comm_wo, send_wo, recv_wo),
            ):
                recv = pltpu.make_async_remote_copy(
                    src_ref=comm.at[s],
                    dst_ref=comm.at[s],
                    send_sem=ssem.at[s],
                    recv_sem=rsem.at[s],
                    device_id=(my_pos,),
                    device_id_type=pl.DeviceIdType.MESH,
                )
                recv.wait_recv()
            if _MODE != "comm":
                origin = lax.rem(my_pos + 1 + s, N_DEV)
                compute(False, origin, comm_wq[s], comm_wo[s],
                        comm_sc[s, 0:1, 0:HD], comm_sc[s, 1:2, :])

        for r in sends:
            r.wait_send()

    return pl.pallas_call(
        body,
        out_shape=jax.ShapeDtypeStruct((B_LOC, SQ, D_MODEL), jnp.float32),
        in_specs=[
            pl.BlockSpec(memory_space=pltpu.VMEM),
            pl.BlockSpec(memory_space=pltpu.VMEM),
            pl.BlockSpec(memory_space=pltpu.VMEM),
            pl.BlockSpec(memory_space=pltpu.VMEM),
            pl.BlockSpec(memory_space=pltpu.VMEM),
        ],
        out_specs=pl.BlockSpec(memory_space=pltpu.VMEM),
        scratch_shapes=[
            pltpu.VMEM((D_MODEL, HD), jnp.int8),
            pltpu.VMEM((HD, D_MODEL), jnp.int8),
            pltpu.VMEM((8, D_MODEL), jnp.bfloat16),
            pltpu.VMEM((N_DEV - 1, D_MODEL, HD), jnp.int8),
            pltpu.VMEM((N_DEV - 1, HD, D_MODEL), jnp.int8),
            pltpu.VMEM((N_DEV - 1, 8, D_MODEL), jnp.bfloat16),
            pltpu.SemaphoreType.DMA((N_DEV - 1,)),
            pltpu.SemaphoreType.DMA((N_DEV - 1,)),
            pltpu.SemaphoreType.DMA((N_DEV - 1,)),
            pltpu.SemaphoreType.DMA((N_DEV - 1,)),
            pltpu.SemaphoreType.DMA((N_DEV - 1,)),
            pltpu.SemaphoreType.DMA((N_DEV - 1,)),
        ],
        compiler_params=pltpu.CompilerParams(
            collective_id=None if _MODE == "compute" else 0
        ),
    )(x, Wq, Wo, k_loc, v_loc)
